# Initial kernel scaffold; baseline (speedup 1.0000x reference)
import jax
import jax.numpy as jnp
from jax.experimental import pallas as pl


def kernel(x, b_prev, prev_b_embedding, w1, b1, w2, b2):
    raise NotImplementedError("write your pallas kernel here")


# speedup vs baseline: 2.6403x; 2.6403x over previous
import jax
import jax.numpy as jnp
import numpy as np
from jax.experimental import pallas as pl
from jax.experimental.pallas import tpu as pltpu

_ALPHA = 300.0
_N_ATTR = 16
_R = 32


def _interp_matrix_t(old: int, new: int) -> np.ndarray:
    pos = np.arange(new, dtype=np.float32) * np.float32((old - 1) / (new - 1))
    lo = np.floor(pos).astype(np.int32)
    hi = np.minimum(lo + 1, old - 1)
    w = pos - lo.astype(np.float32)
    m = np.zeros((new, old), dtype=np.float32)
    m[np.arange(new), lo] += (np.float32(1.0) - w)
    m[np.arange(new), hi] += w
    return np.ascontiguousarray(m.T)


def _softplus(z):
    return jnp.maximum(z, 0.0) + jnp.log1p(jnp.exp(-jnp.abs(z)))


def _fused_kernel(x_ref, emb_ref, bpv_ref, lht_ref, lwt_ref, w1_ref, b1_ref,
                  w2_ref, b2_ref, out_ref, ewt_ref, bwt_ref):
    t = pl.program_id(1)

    @pl.when(t == 0)
    def _prep():
        lwt = lwt_ref[...]
        ew = jnp.dot(emb_ref[0].reshape(256 * 64, 64), lwt,
                     preferred_element_type=jnp.float32)
        ewt_ref[...] = jnp.swapaxes(ew.reshape(256, 64, 128), 1, 2)
        bw = jnp.dot(bpv_ref[0].reshape(64 * 64, 64), lwt,
                     preferred_element_type=jnp.float32)
        bwt_ref[...] = jnp.swapaxes(bw.reshape(64, 64, 128), 1, 2)

    lht = lht_ref[0]

    er = jnp.dot(ewt_ref[...].reshape(256 * 128, 64), lht,
                 preferred_element_type=jnp.float32)
    emb_r = jnp.swapaxes(er.reshape(256, 128, _R), 1, 2)

    xe = x_ref[0] + emb_r

    h1 = jax.lax.dot_general(w1_ref[...], xe, (((1,), (0,)), ((), ())),
                             preferred_element_type=jnp.float32)
    hidd = jnp.maximum(h1 + b1_ref[...], 0.0)
    a1 = jax.lax.dot_general(w2_ref[...], hidd, (((1,), (0,)), ((), ())),
                             preferred_element_type=jnp.float32)
    attr = _softplus(a1 + b2_ref[...])

    bcr = jnp.dot(bwt_ref[...].reshape(64 * 128, 64), lht,
                  preferred_element_type=jnp.float32)
    bc = jnp.swapaxes(bcr.reshape(64, 128, _R), 1, 2)

    acc = bc
    for a in range(_N_ATTR):
        dx = attr[a:a + 1] - bc
        acc = acc + dx / (1.0 + _ALPHA * (dx * dx))
    out_ref[0] = acc


@jax.jit
def kernel(x, b_prev, prev_b_embedding, w1, b1, w2, b2):
    n, c, h, w = x.shape
    nb = b_prev.shape[1]
    md = w1.shape[0]
    na = w2.shape[0]
    grid_t = h // _R

    lht_np = _interp_matrix_t(64, h)
    lht = jnp.asarray(
        np.stack([lht_np[:, t * _R:(t + 1) * _R] for t in range(grid_t)]))
    lwt = jnp.asarray(_interp_matrix_t(64, w))
    b1b = jnp.broadcast_to(b1[:, None, None], (md, 1, w))
    b2b = jnp.broadcast_to(b2[:, None, None], (na, 1, w))

    out = pl.pallas_call(
        _fused_kernel,
        grid=(n, grid_t),
        in_specs=[
            pl.BlockSpec((1, c, _R, w), lambda i, t: (i, 0, t, 0)),
            pl.BlockSpec((1, c, 64, 64), lambda i, t: (i, 0, 0, 0)),
            pl.BlockSpec((1, nb, 64, 64), lambda i, t: (i, 0, 0, 0)),
            pl.BlockSpec((1, 64, _R), lambda i, t: (t, 0, 0)),
            pl.BlockSpec((64, 128), lambda i, t: (0, 0)),
            pl.BlockSpec((md, c), lambda i, t: (0, 0)),
            pl.BlockSpec((md, 1, w), lambda i, t: (0, 0, 0)),
            pl.BlockSpec((na, md), lambda i, t: (0, 0)),
            pl.BlockSpec((na, 1, w), lambda i, t: (0, 0, 0)),
        ],
        out_specs=pl.BlockSpec((1, nb, _R, w), lambda i, t: (i, 0, t, 0)),
        out_shape=jax.ShapeDtypeStruct((n, nb, h, w), jnp.float32),
        scratch_shapes=[
            pltpu.VMEM((c, 128, 64), jnp.float32),
            pltpu.VMEM((nb, 128, 64), jnp.float32),
        ],
        compiler_params=pltpu.CompilerParams(
            dimension_semantics=("parallel", "arbitrary"),
            vmem_limit_bytes=64 * 1024 * 1024,
        ),
    )(x, prev_b_embedding, b_prev, lht, lwt, w1, b1b, w2, b2b)
    return (out, out)
